# baseline (speedup 1.0000x reference)
import jax
import jax.numpy as jnp
from jax.experimental import pallas as pl
from jax.experimental.pallas import tpu as pltpu

N = 4096
F = 64
LAM0 = 1.0
LAM1 = 1.0
LAM4 = 0.0
ALPHA = 1.0 / (1.0 + LAM4 + LAM0 + LAM1)

BLK = 256
GRID = N // BLK


def _s_pass(ab_ref, ag_ref, y_ref, y0_ref, out_ref):
    s = LAM0 * ab_ref[...] + LAM1 * ag_ref[...]
    out_ref[...] = (
        jnp.dot(s, y_ref[...], preferred_element_type=jnp.float32,
                precision=jax.lax.Precision.HIGHEST)
        + y0_ref[...]
    )


def _q_pass_first(db_ref, dg_ref, i_ref, y_ref, yhat_ref, out_ref, qs_ref):
    qs = ALPHA / (LAM0 * db_ref[...] + LAM1 * dg_ref[...] + i_ref[...])
    qs_ref[...] = qs
    out_ref[...] = (1.0 - ALPHA) * y_ref[...] + jnp.dot(
        qs, yhat_ref[...], preferred_element_type=jnp.float32,
        precision=jax.lax.Precision.HIGHEST)


def _q_pass_second(qs_ref, y_ref, yhat_ref, out_ref):
    out_ref[...] = (1.0 - ALPHA) * y_ref[...] + jnp.dot(
        qs_ref[...], yhat_ref[...], preferred_element_type=jnp.float32,
        precision=jax.lax.Precision.HIGHEST)


_mat_spec = pl.BlockSpec((BLK, N), lambda i: (i, 0))
_vec_spec = pl.BlockSpec((BLK, F), lambda i: (i, 0))
_full_spec = pl.BlockSpec((N, F), lambda i: (0, 0))
_params = pltpu.CompilerParams(dimension_semantics=("parallel",))


def kernel(X, A_beta, A_gamma, D_beta, D_gamma, I_mat):
    f32 = jnp.float32

    yhat1 = pl.pallas_call(
        _s_pass,
        grid=(GRID,),
        in_specs=[_mat_spec, _mat_spec, _full_spec, _vec_spec],
        out_specs=_vec_spec,
        out_shape=jax.ShapeDtypeStruct((N, F), f32),
        compiler_params=_params,
    )(A_beta, A_gamma, X, X)

    y1, qs = pl.pallas_call(
        _q_pass_first,
        grid=(GRID,),
        in_specs=[_mat_spec, _mat_spec, _mat_spec, _vec_spec, _full_spec],
        out_specs=(_vec_spec, _mat_spec),
        out_shape=(jax.ShapeDtypeStruct((N, F), f32),
                   jax.ShapeDtypeStruct((N, N), f32)),
        compiler_params=_params,
    )(D_beta, D_gamma, I_mat, X, yhat1)

    yhat2 = pl.pallas_call(
        _s_pass,
        grid=(GRID,),
        in_specs=[_mat_spec, _mat_spec, _full_spec, _vec_spec],
        out_specs=_vec_spec,
        out_shape=jax.ShapeDtypeStruct((N, F), f32),
        compiler_params=_params,
    )(A_beta, A_gamma, y1, X)

    y2 = pl.pallas_call(
        _q_pass_second,
        grid=(GRID,),
        in_specs=[_mat_spec, _vec_spec, _full_spec],
        out_specs=_vec_spec,
        out_shape=jax.ShapeDtypeStruct((N, F), f32),
        compiler_params=_params,
    )(qs, y1, yhat2)

    return y2


# speedup vs baseline: 1.3833x; 1.3833x over previous
import jax
import jax.numpy as jnp
from jax.experimental import pallas as pl
from jax.experimental.pallas import tpu as pltpu

N = 4096
F = 64
LAM0 = 1.0
LAM1 = 1.0
LAM4 = 0.0
ALPHA = 1.0 / (1.0 + LAM4 + LAM0 + LAM1)

BLK = 256
GRID = N // BLK


def _s_pass(ab_ref, ag_ref, y_ref, y0_ref, out_ref):
    s = LAM0 * ab_ref[...] + LAM1 * ag_ref[...]
    out_ref[...] = (
        jnp.dot(s, y_ref[...], preferred_element_type=jnp.float32)
        + y0_ref[...]
    )


def _q_pass_first(db_ref, dg_ref, i_ref, y_ref, yhat_ref, out_ref, qs_ref):
    qs = ALPHA / (LAM0 * db_ref[...] + LAM1 * dg_ref[...] + i_ref[...])
    qs_ref[...] = qs
    out_ref[...] = (1.0 - ALPHA) * y_ref[...] + jnp.dot(
        qs, yhat_ref[...], preferred_element_type=jnp.float32)


def _q_pass_second(qs_ref, y_ref, yhat_ref, out_ref):
    out_ref[...] = (1.0 - ALPHA) * y_ref[...] + jnp.dot(
        qs_ref[...], yhat_ref[...], preferred_element_type=jnp.float32)


_mat_spec = pl.BlockSpec((BLK, N), lambda i: (i, 0))
_vec_spec = pl.BlockSpec((BLK, F), lambda i: (i, 0))
_full_spec = pl.BlockSpec((N, F), lambda i: (0, 0))
_params = pltpu.CompilerParams(dimension_semantics=("parallel",))


def kernel(X, A_beta, A_gamma, D_beta, D_gamma, I_mat):
    f32 = jnp.float32

    yhat1 = pl.pallas_call(
        _s_pass,
        grid=(GRID,),
        in_specs=[_mat_spec, _mat_spec, _full_spec, _vec_spec],
        out_specs=_vec_spec,
        out_shape=jax.ShapeDtypeStruct((N, F), f32),
        compiler_params=_params,
    )(A_beta, A_gamma, X, X)

    y1, qs = pl.pallas_call(
        _q_pass_first,
        grid=(GRID,),
        in_specs=[_mat_spec, _mat_spec, _mat_spec, _vec_spec, _full_spec],
        out_specs=(_vec_spec, _mat_spec),
        out_shape=(jax.ShapeDtypeStruct((N, F), f32),
                   jax.ShapeDtypeStruct((N, N), f32)),
        compiler_params=_params,
    )(D_beta, D_gamma, I_mat, X, yhat1)

    yhat2 = pl.pallas_call(
        _s_pass,
        grid=(GRID,),
        in_specs=[_mat_spec, _mat_spec, _full_spec, _vec_spec],
        out_specs=_vec_spec,
        out_shape=jax.ShapeDtypeStruct((N, F), f32),
        compiler_params=_params,
    )(A_beta, A_gamma, y1, X)

    y2 = pl.pallas_call(
        _q_pass_second,
        grid=(GRID,),
        in_specs=[_mat_spec, _vec_spec, _full_spec],
        out_specs=_vec_spec,
        out_shape=jax.ShapeDtypeStruct((N, F), f32),
        compiler_params=_params,
    )(qs, y1, yhat2)

    return y2


# speedup vs baseline: 1.5947x; 1.1529x over previous
import jax
import jax.numpy as jnp
from jax.experimental import pallas as pl
from jax.experimental.pallas import tpu as pltpu

N = 4096
F = 64
LAM0 = 1.0
LAM1 = 1.0
LAM4 = 0.0
ALPHA = 1.0 / (1.0 + LAM4 + LAM0 + LAM1)

BLK = 256
GRID = N // BLK


def _s_pass_first(ab_ref, ag_ref, y_ref, y0_ref, out_ref, s_ref):
    s = LAM0 * ab_ref[...] + LAM1 * ag_ref[...]
    s_ref[...] = s.astype(jnp.bfloat16)
    out_ref[...] = (
        jnp.dot(s, y_ref[...], preferred_element_type=jnp.float32)
        + y0_ref[...]
    )


def _s_pass_second(s_ref, y_ref, y0_ref, out_ref):
    s = s_ref[...].astype(jnp.float32)
    out_ref[...] = (
        jnp.dot(s, y_ref[...], preferred_element_type=jnp.float32)
        + y0_ref[...]
    )


def _q_pass_first(db_ref, dg_ref, i_ref, y_ref, yhat_ref, out_ref, qs_ref):
    qs = ALPHA / (LAM0 * db_ref[...] + LAM1 * dg_ref[...] + i_ref[...])
    qs_ref[...] = qs.astype(jnp.bfloat16)
    out_ref[...] = (1.0 - ALPHA) * y_ref[...] + jnp.dot(
        qs, yhat_ref[...], preferred_element_type=jnp.float32)


def _q_pass_second(qs_ref, y_ref, yhat_ref, out_ref):
    qs = qs_ref[...].astype(jnp.float32)
    out_ref[...] = (1.0 - ALPHA) * y_ref[...] + jnp.dot(
        qs, yhat_ref[...], preferred_element_type=jnp.float32)


_mat_spec = pl.BlockSpec((BLK, N), lambda i: (i, 0))
_vec_spec = pl.BlockSpec((BLK, F), lambda i: (i, 0))
_full_spec = pl.BlockSpec((N, F), lambda i: (0, 0))
_params = pltpu.CompilerParams(dimension_semantics=("parallel",))


def kernel(X, A_beta, A_gamma, D_beta, D_gamma, I_mat):
    f32 = jnp.float32

    yhat1, s_bf16 = pl.pallas_call(
        _s_pass_first,
        grid=(GRID,),
        in_specs=[_mat_spec, _mat_spec, _full_spec, _vec_spec],
        out_specs=(_vec_spec, _mat_spec),
        out_shape=(jax.ShapeDtypeStruct((N, F), f32),
                   jax.ShapeDtypeStruct((N, N), jnp.bfloat16)),
        compiler_params=_params,
    )(A_beta, A_gamma, X, X)

    y1, qs = pl.pallas_call(
        _q_pass_first,
        grid=(GRID,),
        in_specs=[_mat_spec, _mat_spec, _mat_spec, _vec_spec, _full_spec],
        out_specs=(_vec_spec, _mat_spec),
        out_shape=(jax.ShapeDtypeStruct((N, F), f32),
                   jax.ShapeDtypeStruct((N, N), jnp.bfloat16)),
        compiler_params=_params,
    )(D_beta, D_gamma, I_mat, X, yhat1)

    yhat2 = pl.pallas_call(
        _s_pass_second,
        grid=(GRID,),
        in_specs=[_mat_spec, _full_spec, _vec_spec],
        out_specs=_vec_spec,
        out_shape=jax.ShapeDtypeStruct((N, F), f32),
        compiler_params=_params,
    )(s_bf16, y1, X)

    y2 = pl.pallas_call(
        _q_pass_second,
        grid=(GRID,),
        in_specs=[_mat_spec, _vec_spec, _full_spec],
        out_specs=_vec_spec,
        out_shape=jax.ShapeDtypeStruct((N, F), f32),
        compiler_params=_params,
    )(qs, y1, yhat2)

    return y2


# speedup vs baseline: 1.6839x; 1.0559x over previous
import jax
import jax.numpy as jnp
from jax.experimental import pallas as pl
from jax.experimental.pallas import tpu as pltpu

N = 4096
F = 64
LAM0 = 1.0
LAM1 = 1.0
LAM4 = 0.0
ALPHA = 1.0 / (1.0 + LAM4 + LAM0 + LAM1)

BLK1 = 512
BLK2 = 256
BLK3 = 512


def _s_pass_first(ab_ref, ag_ref, y_ref, y0_ref, out_ref, s_ref):
    s = LAM0 * ab_ref[...] + LAM1 * ag_ref[...]
    s_ref[...] = s.astype(jnp.bfloat16)
    out_ref[...] = (
        jnp.dot(s, y_ref[...], preferred_element_type=jnp.float32)
        + y0_ref[...]
    )


def _q_pass_first(db_ref, dg_ref, i_ref, y_ref, yhat_ref, out_ref, qs_ref):
    qs = ALPHA / (LAM0 * db_ref[...] + LAM1 * dg_ref[...] + i_ref[...])
    qs_ref[...] = qs.astype(jnp.bfloat16)
    out_ref[...] = (1.0 - ALPHA) * y_ref[...] + jnp.dot(
        qs, yhat_ref[...], preferred_element_type=jnp.float32)


def _step2_fused(s_ref, qs_ref, y1_ref, x_ref, out_ref, yhat_ref):
    p = pl.program_id(0)
    i = pl.program_id(1)
    rows = pl.ds(i * BLK3, BLK3)

    @pl.when(p == 0)
    def _():
        s = s_ref[...].astype(jnp.float32)
        yhat_ref[rows, :] = (
            jnp.dot(s, y1_ref[...], preferred_element_type=jnp.float32)
            + x_ref[...]
        )

    @pl.when(p == 1)
    def _():
        qs = qs_ref[...].astype(jnp.float32)
        out_ref[...] = (1.0 - ALPHA) * y1_ref[rows, :] + jnp.dot(
            qs, yhat_ref[...], preferred_element_type=jnp.float32)


_params = pltpu.CompilerParams(dimension_semantics=("parallel",))


def kernel(X, A_beta, A_gamma, D_beta, D_gamma, I_mat):
    f32 = jnp.float32
    bf16 = jnp.bfloat16

    yhat1, s_bf16 = pl.pallas_call(
        _s_pass_first,
        grid=(N // BLK1,),
        in_specs=[
            pl.BlockSpec((BLK1, N), lambda i: (i, 0)),
            pl.BlockSpec((BLK1, N), lambda i: (i, 0)),
            pl.BlockSpec((N, F), lambda i: (0, 0)),
            pl.BlockSpec((BLK1, F), lambda i: (i, 0)),
        ],
        out_specs=(pl.BlockSpec((BLK1, F), lambda i: (i, 0)),
                   pl.BlockSpec((BLK1, N), lambda i: (i, 0))),
        out_shape=(jax.ShapeDtypeStruct((N, F), f32),
                   jax.ShapeDtypeStruct((N, N), bf16)),
        compiler_params=_params,
    )(A_beta, A_gamma, X, X)

    y1, qs = pl.pallas_call(
        _q_pass_first,
        grid=(N // BLK2,),
        in_specs=[
            pl.BlockSpec((BLK2, N), lambda i: (i, 0)),
            pl.BlockSpec((BLK2, N), lambda i: (i, 0)),
            pl.BlockSpec((BLK2, N), lambda i: (i, 0)),
            pl.BlockSpec((BLK2, F), lambda i: (i, 0)),
            pl.BlockSpec((N, F), lambda i: (0, 0)),
        ],
        out_specs=(pl.BlockSpec((BLK2, F), lambda i: (i, 0)),
                   pl.BlockSpec((BLK2, N), lambda i: (i, 0))),
        out_shape=(jax.ShapeDtypeStruct((N, F), f32),
                   jax.ShapeDtypeStruct((N, N), bf16)),
        compiler_params=_params,
    )(D_beta, D_gamma, I_mat, X, yhat1)

    y2 = pl.pallas_call(
        _step2_fused,
        grid=(2, N // BLK3),
        in_specs=[
            pl.BlockSpec((BLK3, N), lambda p, i: (i * (1 - p), 0)),
            pl.BlockSpec((BLK3, N), lambda p, i: (i * p, 0)),
            pl.BlockSpec((N, F), lambda p, i: (0, 0)),
            pl.BlockSpec((BLK3, F), lambda p, i: (i, 0)),
        ],
        out_specs=pl.BlockSpec((BLK3, F), lambda p, i: (i, 0)),
        out_shape=jax.ShapeDtypeStruct((N, F), f32),
        scratch_shapes=[pltpu.VMEM((N, F), f32)],
        compiler_params=pltpu.CompilerParams(
            dimension_semantics=("arbitrary", "arbitrary")),
    )(s_bf16, qs, y1, X)

    return y2
